# baseline (speedup 1.0000x reference)
import functools

import numpy as np
import jax
import jax.numpy as jnp
from jax import lax
from jax.experimental import pallas as pl
from jax.experimental.pallas import tpu as pltpu

_STEPS = 10
_BIG = np.float32(1e9)
_TINY = np.float32(np.finfo(np.float32).tiny)


def _rotl(x, r):
    return lax.shift_left(x, np.uint32(r)) | lax.shift_right_logical(
        x, np.uint32(32 - r))


def _threefry(k1, k2, x0, x1):
    ks2 = k1 ^ k2 ^ np.uint32(0x1BD11BDA)
    x = [x0 + k1, x1 + k2]

    def rnds(x, rs):
        for r in rs:
            x[0] = x[0] + x[1]
            x[1] = _rotl(x[1], r)
            x[1] = x[0] ^ x[1]
        return x

    r0 = (13, 15, 26, 6)
    r1 = (17, 29, 16, 24)
    x = rnds(x, r0)
    x = [x[0] + k2, x[1] + ks2 + np.uint32(1)]
    x = rnds(x, r1)
    x = [x[0] + ks2, x[1] + k1 + np.uint32(2)]
    x = rnds(x, r0)
    x = [x[0] + k1, x[1] + k2 + np.uint32(3)]
    x = rnds(x, r1)
    x = [x[0] + k2, x[1] + ks2 + np.uint32(4)]
    x = rnds(x, r0)
    x = [x[0] + ks2, x[1] + k1 + np.uint32(5)]
    return x[0], x[1]


def _bits_to_unit(bits):
    fb = lax.shift_right_logical(bits, np.uint32(9)) | np.uint32(0x3F800000)
    return lax.bitcast_convert_type(fb, jnp.float32) - np.float32(1.0)


def _encode_body(x_ref, o_ref, *, K):
    kio = lax.broadcasted_iota(jnp.int32, x_ref.shape, 2).astype(jnp.float32)
    cf = jnp.sum(x_ref[...] * kio, axis=2)
    o_ref[...] = cf.astype(jnp.int32).T


def _decode_body(c_ref, o_ref, *, K):
    ct = c_ref[...].T
    kio = lax.broadcasted_iota(jnp.int32, o_ref.shape, 2)
    o_ref[...] = (kio == ct[:, :, None]).astype(jnp.float32)


def _main_body(ks_ref, ku_ref, theta_ref, cur0_ref, out_ref,
               thc_ref, ef_ref, bv_ref, bi_ref, bfl_ref, sf_ref,
               *, B, D, K, DT, NT):
    s = pl.program_id(0)
    t = pl.program_id(1)

    @pl.when((s == 0) & (t == 0))
    def _init():
        out_ref[...] = cur0_ref[...]

    def finalize(step):
        idx = bi_ref[0:1, :]
        dstar = idx // K
        knew = idx - dstar * K
        diota = lax.broadcasted_iota(jnp.int32, (D, B), 0)
        dmask = diota == dstar
        th_old = jnp.sum(jnp.where(dmask, thc_ref[...], 0.0), axis=0,
                         keepdims=True)
        e_d = jnp.sum(jnp.where(dmask, ef_ref[...], 0.0), axis=0,
                      keepdims=True)
        th_new = th_old + 2.0 * bfl_ref[0:1, :]
        c_old = jnp.exp(-0.5 * th_old) * (e_d - jnp.exp(0.5 * th_old))
        c_new = jnp.exp(-0.5 * th_new) * (e_d - jnp.exp(0.5 * th_new))
        s_f = sf_ref[0:1, :]
        s_r = s_f - c_old + c_new
        la = jnp.log(s_f) - jnp.log(s_r)
        ju = lax.broadcasted_iota(jnp.uint32, (1, B), 1)
        a0, a1 = _threefry(ku_ref[step, 0], ku_ref[step, 1],
                           jnp.zeros_like(ju), ju)
        u = jnp.maximum(_bits_to_unit(a0 ^ a1), np.float32(0.0))
        acc = jnp.exp(la) > u
        out_ref[...] = jnp.where(dmask & acc, knew, out_ref[...])

    @pl.when((t == 0) & (s > 0))
    def _fin_prev():
        finalize(s - 1)

    @pl.when(t == 0)
    def _reset():
        bv_ref[...] = jnp.full_like(bv_ref[...], -jnp.inf)
        bi_ref[...] = jnp.zeros_like(bi_ref[...])
        bfl_ref[...] = jnp.zeros_like(bfl_ref[...])
        sf_ref[...] = jnp.zeros_like(sf_ref[...])

    U = min(8, DT)
    R = U * K
    d0 = t * DT
    k1 = ks_ref[s, 0]
    k2 = ks_ref[s, 1]
    kio3 = lax.broadcasted_iota(jnp.int32, (U, K, B), 1)
    bio3 = lax.broadcasted_iota(jnp.uint32, (U, K, B), 2)
    uio3 = lax.broadcasted_iota(jnp.uint32, (U, K, B), 0)
    base = (bio3 * np.uint32(D * K) + uio3 * np.uint32(K)
            + lax.convert_element_type(kio3, jnp.uint32))
    dio_u = lax.broadcasted_iota(jnp.int32, (U, B), 0)

    def chunk(c, carry):
        bv, bi, bfl, sf = carry
        d = d0 + c * U
        curb = out_ref[pl.ds(d, U), :]
        th_blk = theta_ref[pl.ds(d, U), :]
        th3 = jnp.broadcast_to(th_blk[:, :, None], (U, K, B))
        maskf = (kio3 == curb[:, None, :]).astype(jnp.float32)
        thc_u = jnp.sum(th3 * maskf, axis=1)
        thc3 = thc_u[:, None, :]
        fl3 = 0.5 * th3 - 0.5 * thc3 - _BIG * maskf
        j = base + lax.convert_element_type(d * K, jnp.uint32)
        o0, o1 = _threefry(k1, k2, jnp.zeros((U, K, B), jnp.uint32), j)
        uu = jnp.maximum(_bits_to_unit(o0 ^ o1), _TINY)
        z = fl3 + (-jnp.log(-jnp.log(uu)))
        zm_u = jnp.max(z, axis=1)
        zm3 = zm_u[:, None, :]
        kk_u = jnp.min(jnp.where(z == zm3, kio3, K), axis=1)
        flz_u = jnp.sum(jnp.where(kio3 == kk_u[:, None, :], fl3, 0.0), axis=1)
        e_u = jnp.sum(jnp.exp(0.5 * th3), axis=1)
        w_u = jnp.exp(-0.5 * thc_u) * (e_u - jnp.exp(0.5 * thc_u))
        sf = sf + jnp.sum(w_u, axis=0, keepdims=True)
        thc_ref[pl.ds(d, U), :] = thc_u
        ef_ref[pl.ds(d, U), :] = e_u
        zmc = jnp.max(zm_u, axis=0, keepdims=True)
        dsel = dio_u == jnp.min(jnp.where(zm_u == zmc, dio_u, U), axis=0,
                                keepdims=True)
        kkc = jnp.sum(jnp.where(dsel, kk_u, 0), axis=0, keepdims=True)
        flc = jnp.sum(jnp.where(dsel, flz_u, 0.0), axis=0, keepdims=True)
        ddc = jnp.sum(jnp.where(dsel, dio_u, 0), axis=0, keepdims=True)
        upd = zmc > bv
        bv = jnp.where(upd, zmc, bv)
        bi = jnp.where(upd, (d + ddc) * K + kkc, bi)
        bfl = jnp.where(upd, flc, bfl)
        return bv, bi, bfl, sf

    ninf = jnp.full((1, B), -jnp.inf, jnp.float32)
    zf = jnp.zeros((1, B), jnp.float32)
    zi = jnp.zeros((1, B), jnp.int32)
    bv_t, bi_t, bfl_t, sf_t = lax.fori_loop(0, DT // U, chunk,
                                            (ninf, zi, zf, zf))

    upd = bv_t > bv_ref[0:1, :]
    bv_ref[0:1, :] = jnp.where(upd, bv_t, bv_ref[0:1, :])
    bi_ref[0:1, :] = jnp.where(upd, bi_t, bi_ref[0:1, :])
    bfl_ref[0:1, :] = jnp.where(upd, bfl_t, bfl_ref[0:1, :])
    sf_ref[0:1, :] = sf_ref[0:1, :] + sf_t

    @pl.when((s == _STEPS - 1) & (t == NT - 1))
    def _fin_last():
        finalize(s)


def kernel(x, theta):
    B, D, K = x.shape
    DT = min(128, D)
    NT = D // DT

    key = jax.random.key(42)
    ksl, kul = [], []
    for _ in range(_STEPS):
        key, ks, ku = jax.random.split(key, 3)
        ksl.append(jax.random.key_data(ks))
        kul.append(jax.random.key_data(ku))
    ksd = jnp.stack(ksl).astype(jnp.uint32)
    kud = jnp.stack(kul).astype(jnp.uint32)

    cur0 = pl.pallas_call(
        functools.partial(_encode_body, K=K),
        grid=(NT,),
        in_specs=[pl.BlockSpec((B, DT, K), lambda t: (0, t, 0))],
        out_specs=pl.BlockSpec((DT, B), lambda t: (t, 0)),
        out_shape=jax.ShapeDtypeStruct((D, B), jnp.int32),
    )(x)

    cur = pl.pallas_call(
        functools.partial(_main_body, B=B, D=D, K=K, DT=DT, NT=NT),
        grid=(_STEPS, NT),
        in_specs=[
            pl.BlockSpec(memory_space=pltpu.SMEM),
            pl.BlockSpec(memory_space=pltpu.SMEM),
            pl.BlockSpec((D, K), lambda s, t: (0, 0)),
            pl.BlockSpec((D, B), lambda s, t: (0, 0)),
        ],
        out_specs=pl.BlockSpec((D, B), lambda s, t: (0, 0)),
        out_shape=jax.ShapeDtypeStruct((D, B), jnp.int32),
        scratch_shapes=[
            pltpu.VMEM((D, B), jnp.float32),
            pltpu.VMEM((D, B), jnp.float32),
            pltpu.VMEM((8, B), jnp.float32),
            pltpu.VMEM((8, B), jnp.int32),
            pltpu.VMEM((8, B), jnp.float32),
            pltpu.VMEM((8, B), jnp.float32),
        ],
    )(ksd, kud, theta, cur0)

    x_out = pl.pallas_call(
        functools.partial(_decode_body, K=K),
        grid=(NT,),
        in_specs=[pl.BlockSpec((DT, B), lambda t: (t, 0))],
        out_specs=pl.BlockSpec((B, DT, K), lambda t: (0, t, 0)),
        out_shape=jax.ShapeDtypeStruct((B, D, K), jnp.float32),
    )(cur)

    return x_out


# speedup vs baseline: 1.0418x; 1.0418x over previous
import functools

import numpy as np
import jax
import jax.numpy as jnp
from jax import lax
from jax.experimental import pallas as pl
from jax.experimental.pallas import tpu as pltpu

_STEPS = 10
_BIG = np.float32(1e9)
_TINY = np.float32(np.finfo(np.float32).tiny)


def _rotl(x, r):
    return lax.shift_left(x, np.uint32(r)) | lax.shift_right_logical(
        x, np.uint32(32 - r))


def _threefry(k1, k2, x0, x1):
    ks2 = k1 ^ k2 ^ np.uint32(0x1BD11BDA)
    x = [x0 + k1, x1 + k2]

    def rnds(x, rs):
        for r in rs:
            x[0] = x[0] + x[1]
            x[1] = _rotl(x[1], r)
            x[1] = x[0] ^ x[1]
        return x

    r0 = (13, 15, 26, 6)
    r1 = (17, 29, 16, 24)
    x = rnds(x, r0)
    x = [x[0] + k2, x[1] + ks2 + np.uint32(1)]
    x = rnds(x, r1)
    x = [x[0] + ks2, x[1] + k1 + np.uint32(2)]
    x = rnds(x, r0)
    x = [x[0] + k1, x[1] + k2 + np.uint32(3)]
    x = rnds(x, r1)
    x = [x[0] + k2, x[1] + ks2 + np.uint32(4)]
    x = rnds(x, r0)
    x = [x[0] + ks2, x[1] + k1 + np.uint32(5)]
    return x[0], x[1]


def _bits_to_unit(bits):
    fb = lax.shift_right_logical(bits, np.uint32(9)) | np.uint32(0x3F800000)
    return lax.bitcast_convert_type(fb, jnp.float32) - np.float32(1.0)


def _encode_body(x_ref, o_ref, *, K):
    kio = lax.broadcasted_iota(jnp.int32, x_ref.shape, 2).astype(jnp.float32)
    cf = jnp.sum(x_ref[...] * kio, axis=2)
    o_ref[...] = cf.astype(jnp.int32).T


def _decode_body(c_ref, o_ref, *, K):
    ct = c_ref[...].T
    kio = lax.broadcasted_iota(jnp.int32, o_ref.shape, 2)
    o_ref[...] = (kio == ct[:, :, None]).astype(jnp.float32)


def _main_body(ks_ref, ku_ref, theta_ref, cur0_ref, out_ref,
               thc_ref, ef_ref, bv_ref, bi_ref, bfl_ref, sf_ref,
               *, B, D, K, DT, NT):
    s = pl.program_id(0)
    t = pl.program_id(1)

    @pl.when((s == 0) & (t == 0))
    def _init():
        out_ref[...] = cur0_ref[...]

    def finalize(step):
        idx = bi_ref[0:1, :]
        dstar = idx // K
        knew = idx - dstar * K
        diota = lax.broadcasted_iota(jnp.int32, (D, B), 0)
        dmask = diota == dstar
        th_old = jnp.sum(jnp.where(dmask, thc_ref[...], 0.0), axis=0,
                         keepdims=True)
        e_d = jnp.sum(jnp.where(dmask, ef_ref[...], 0.0), axis=0,
                      keepdims=True)
        th_new = th_old + 2.0 * bfl_ref[0:1, :]
        c_old = jnp.exp(-0.5 * th_old) * (e_d - jnp.exp(0.5 * th_old))
        c_new = jnp.exp(-0.5 * th_new) * (e_d - jnp.exp(0.5 * th_new))
        s_f = sf_ref[0:1, :]
        s_r = s_f - c_old + c_new
        la = jnp.log(s_f) - jnp.log(s_r)
        ju = lax.broadcasted_iota(jnp.uint32, (1, B), 1)
        a0, a1 = _threefry(ku_ref[step, 0], ku_ref[step, 1],
                           jnp.zeros_like(ju), ju)
        u = jnp.maximum(_bits_to_unit(a0 ^ a1), np.float32(0.0))
        acc = jnp.exp(la) > u
        out_ref[...] = jnp.where(dmask & acc, knew, out_ref[...])

    @pl.when((t == 0) & (s > 0))
    def _fin_prev():
        finalize(s - 1)

    @pl.when(t == 0)
    def _reset():
        bv_ref[...] = jnp.full_like(bv_ref[...], -jnp.inf)
        bi_ref[...] = jnp.zeros_like(bi_ref[...])
        bfl_ref[...] = jnp.zeros_like(bfl_ref[...])
        sf_ref[...] = jnp.zeros_like(sf_ref[...])

    U = min(2, DT)
    R = U * K
    d0 = t * DT
    k1 = ks_ref[s, 0]
    k2 = ks_ref[s, 1]
    kio3 = lax.broadcasted_iota(jnp.int32, (U, K, B), 1)
    bio3 = lax.broadcasted_iota(jnp.uint32, (U, K, B), 2)
    uio3 = lax.broadcasted_iota(jnp.uint32, (U, K, B), 0)
    base = (bio3 * np.uint32(D * K) + uio3 * np.uint32(K)
            + lax.convert_element_type(kio3, jnp.uint32))
    dio_u = lax.broadcasted_iota(jnp.int32, (U, B), 0)

    def chunk(c, carry):
        bv, bi, bfl, sf = carry
        d = d0 + c * U
        curb = out_ref[pl.ds(d, U), :]
        th_blk = theta_ref[pl.ds(d, U), :]
        th3 = jnp.broadcast_to(th_blk[:, :, None], (U, K, B))
        maskf = (kio3 == curb[:, None, :]).astype(jnp.float32)
        thc_u = jnp.sum(th3 * maskf, axis=1)
        thc3 = thc_u[:, None, :]
        fl3 = 0.5 * th3 - 0.5 * thc3 - _BIG * maskf
        j = base + lax.convert_element_type(d * K, jnp.uint32)
        o0, o1 = _threefry(k1, k2, jnp.zeros((U, K, B), jnp.uint32), j)
        uu = jnp.maximum(_bits_to_unit(o0 ^ o1), _TINY)
        z = fl3 + (-jnp.log(-jnp.log(uu)))
        zm_u = jnp.max(z, axis=1)
        zm3 = zm_u[:, None, :]
        kk_u = jnp.min(jnp.where(z == zm3, kio3, K), axis=1)
        flz_u = jnp.sum(jnp.where(kio3 == kk_u[:, None, :], fl3, 0.0), axis=1)
        e_u = jnp.sum(jnp.exp(0.5 * th3), axis=1)
        w_u = jnp.exp(-0.5 * thc_u) * (e_u - jnp.exp(0.5 * thc_u))
        sf = sf + jnp.sum(w_u, axis=0, keepdims=True)
        thc_ref[pl.ds(d, U), :] = thc_u
        ef_ref[pl.ds(d, U), :] = e_u
        zmc = jnp.max(zm_u, axis=0, keepdims=True)
        dsel = dio_u == jnp.min(jnp.where(zm_u == zmc, dio_u, U), axis=0,
                                keepdims=True)
        kkc = jnp.sum(jnp.where(dsel, kk_u, 0), axis=0, keepdims=True)
        flc = jnp.sum(jnp.where(dsel, flz_u, 0.0), axis=0, keepdims=True)
        ddc = jnp.sum(jnp.where(dsel, dio_u, 0), axis=0, keepdims=True)
        upd = zmc > bv
        bv = jnp.where(upd, zmc, bv)
        bi = jnp.where(upd, (d + ddc) * K + kkc, bi)
        bfl = jnp.where(upd, flc, bfl)
        return bv, bi, bfl, sf

    ninf = jnp.full((1, B), -jnp.inf, jnp.float32)
    zf = jnp.zeros((1, B), jnp.float32)
    zi = jnp.zeros((1, B), jnp.int32)
    bv_t, bi_t, bfl_t, sf_t = lax.fori_loop(0, DT // U, chunk,
                                            (ninf, zi, zf, zf))

    upd = bv_t > bv_ref[0:1, :]
    bv_ref[0:1, :] = jnp.where(upd, bv_t, bv_ref[0:1, :])
    bi_ref[0:1, :] = jnp.where(upd, bi_t, bi_ref[0:1, :])
    bfl_ref[0:1, :] = jnp.where(upd, bfl_t, bfl_ref[0:1, :])
    sf_ref[0:1, :] = sf_ref[0:1, :] + sf_t

    @pl.when((s == _STEPS - 1) & (t == NT - 1))
    def _fin_last():
        finalize(s)


def kernel(x, theta):
    B, D, K = x.shape
    DT = min(128, D)
    NT = D // DT

    key = jax.random.key(42)
    ksl, kul = [], []
    for _ in range(_STEPS):
        key, ks, ku = jax.random.split(key, 3)
        ksl.append(jax.random.key_data(ks))
        kul.append(jax.random.key_data(ku))
    ksd = jnp.stack(ksl).astype(jnp.uint32)
    kud = jnp.stack(kul).astype(jnp.uint32)

    cur0 = pl.pallas_call(
        functools.partial(_encode_body, K=K),
        grid=(NT,),
        in_specs=[pl.BlockSpec((B, DT, K), lambda t: (0, t, 0))],
        out_specs=pl.BlockSpec((DT, B), lambda t: (t, 0)),
        out_shape=jax.ShapeDtypeStruct((D, B), jnp.int32),
    )(x)

    cur = pl.pallas_call(
        functools.partial(_main_body, B=B, D=D, K=K, DT=DT, NT=NT),
        grid=(_STEPS, NT),
        in_specs=[
            pl.BlockSpec(memory_space=pltpu.SMEM),
            pl.BlockSpec(memory_space=pltpu.SMEM),
            pl.BlockSpec((D, K), lambda s, t: (0, 0)),
            pl.BlockSpec((D, B), lambda s, t: (0, 0)),
        ],
        out_specs=pl.BlockSpec((D, B), lambda s, t: (0, 0)),
        out_shape=jax.ShapeDtypeStruct((D, B), jnp.int32),
        scratch_shapes=[
            pltpu.VMEM((D, B), jnp.float32),
            pltpu.VMEM((D, B), jnp.float32),
            pltpu.VMEM((8, B), jnp.float32),
            pltpu.VMEM((8, B), jnp.int32),
            pltpu.VMEM((8, B), jnp.float32),
            pltpu.VMEM((8, B), jnp.float32),
        ],
    )(ksd, kud, theta, cur0)

    x_out = pl.pallas_call(
        functools.partial(_decode_body, K=K),
        grid=(NT,),
        in_specs=[pl.BlockSpec((DT, B), lambda t: (t, 0))],
        out_specs=pl.BlockSpec((B, DT, K), lambda t: (0, t, 0)),
        out_shape=jax.ShapeDtypeStruct((B, D, K), jnp.float32),
    )(cur)

    return x_out


# speedup vs baseline: 1.1419x; 1.0960x over previous
import functools

import numpy as np
import jax
import jax.numpy as jnp
from jax import lax
from jax.experimental import pallas as pl
from jax.experimental.pallas import tpu as pltpu

_STEPS = 10
_BIG = np.float32(1e9)
_TINY = np.float32(np.finfo(np.float32).tiny)


def _rotl(x, r):
    return lax.shift_left(x, np.uint32(r)) | lax.shift_right_logical(
        x, np.uint32(32 - r))


def _threefry(k1, k2, x0, x1):
    ks2 = k1 ^ k2 ^ np.uint32(0x1BD11BDA)
    x = [x0 + k1, x1 + k2]

    def rnds(x, rs):
        for r in rs:
            x[0] = x[0] + x[1]
            x[1] = _rotl(x[1], r)
            x[1] = x[0] ^ x[1]
        return x

    r0 = (13, 15, 26, 6)
    r1 = (17, 29, 16, 24)
    x = rnds(x, r0)
    x = [x[0] + k2, x[1] + ks2 + np.uint32(1)]
    x = rnds(x, r1)
    x = [x[0] + ks2, x[1] + k1 + np.uint32(2)]
    x = rnds(x, r0)
    x = [x[0] + k1, x[1] + k2 + np.uint32(3)]
    x = rnds(x, r1)
    x = [x[0] + k2, x[1] + ks2 + np.uint32(4)]
    x = rnds(x, r0)
    x = [x[0] + ks2, x[1] + k1 + np.uint32(5)]
    return x[0], x[1]


def _bits_to_unit(bits):
    fb = lax.shift_right_logical(bits, np.uint32(9)) | np.uint32(0x3F800000)
    return lax.bitcast_convert_type(fb, jnp.float32) - np.float32(1.0)


def _encode_body(x_ref, o_ref, *, K):
    kio = lax.broadcasted_iota(jnp.int32, x_ref.shape, 2).astype(jnp.float32)
    cf = jnp.sum(x_ref[...] * kio, axis=2)
    o_ref[...] = cf.astype(jnp.int32).T


def _decode_body(c_ref, o_ref, *, K):
    ct = c_ref[...].T
    kio = lax.broadcasted_iota(jnp.int32, o_ref.shape, 2)
    o_ref[...] = (kio == ct[:, :, None]).astype(jnp.float32)


def _main_body(ks_ref, ku_ref, theta_ref, cur0_ref, out_ref,
               thc_ref, ef_ref, bv_ref, bi_ref, bfl_ref, sf_ref, thh_ref,
               *, B, D, K, DT, NT):
    s = pl.program_id(0)
    t = pl.program_id(1)

    @pl.when((s == 0) & (t == 0))
    def _init():
        out_ref[...] = cur0_ref[...]
        hth = 0.5 * theta_ref[...]
        thh_ref[...] = hth
        e_col = jnp.sum(jnp.exp(hth), axis=1, keepdims=True)
        ef_ref[...] = jnp.broadcast_to(e_col, (D, B))

    def finalize(step):
        idx = bi_ref[0:1, :]
        dstar = idx // K
        knew = idx - dstar * K
        diota = lax.broadcasted_iota(jnp.int32, (D, B), 0)
        dmask = diota == dstar
        hth_old = jnp.sum(jnp.where(dmask, thc_ref[...], 0.0), axis=0,
                          keepdims=True)
        e_d = jnp.sum(jnp.where(dmask, ef_ref[...], 0.0), axis=0,
                      keepdims=True)
        hth_new = hth_old + bfl_ref[0:1, :]
        c_old = jnp.exp(-hth_old) * (e_d - jnp.exp(hth_old))
        c_new = jnp.exp(-hth_new) * (e_d - jnp.exp(hth_new))
        s_f = sf_ref[0:1, :]
        s_r = s_f - c_old + c_new
        la = jnp.log(s_f) - jnp.log(s_r)
        ju = lax.broadcasted_iota(jnp.uint32, (1, B), 1)
        a0, a1 = _threefry(ku_ref[step, 0], ku_ref[step, 1],
                           jnp.zeros_like(ju), ju)
        u = jnp.maximum(_bits_to_unit(a0 ^ a1), np.float32(0.0))
        acc = jnp.exp(la) > u
        out_ref[...] = jnp.where(dmask & acc, knew, out_ref[...])

    @pl.when((t == 0) & (s > 0))
    def _fin_prev():
        finalize(s - 1)

    @pl.when(t == 0)
    def _reset():
        bv_ref[...] = jnp.full_like(bv_ref[...], -jnp.inf)
        bi_ref[...] = jnp.zeros_like(bi_ref[...])
        bfl_ref[...] = jnp.zeros_like(bfl_ref[...])
        sf_ref[...] = jnp.zeros_like(sf_ref[...])

    U = min(4, DT)
    R = U * K
    d0 = t * DT
    k1 = ks_ref[s, 0]
    k2 = ks_ref[s, 1]
    kio3 = lax.broadcasted_iota(jnp.int32, (U, K, B), 1)
    bio3 = lax.broadcasted_iota(jnp.uint32, (U, K, B), 2)
    uio3 = lax.broadcasted_iota(jnp.uint32, (U, K, B), 0)
    base = (bio3 * np.uint32(D * K) + uio3 * np.uint32(K)
            + lax.convert_element_type(kio3, jnp.uint32))
    dio_u = lax.broadcasted_iota(jnp.int32, (U, B), 0)

    def chunk(c, carry):
        bv, bi, bfl, sf = carry
        d = d0 + c * U
        curb = out_ref[pl.ds(d, U), :]
        th_blk = thh_ref[pl.ds(d, U), :]
        th3 = jnp.broadcast_to(th_blk[:, :, None], (U, K, B))
        maskf = (kio3 == curb[:, None, :]).astype(jnp.float32)
        thc_u = jnp.sum(th3 * maskf, axis=1)
        fl3 = th3 - thc_u[:, None, :] - _BIG * maskf
        j = base + lax.convert_element_type(d * K, jnp.uint32)
        o0, o1 = _threefry(k1, k2, jnp.zeros((U, K, B), jnp.uint32), j)
        uu = jnp.maximum(_bits_to_unit(o0 ^ o1), _TINY)
        z = fl3 + (-jnp.log(-jnp.log(uu)))
        zm_u = jnp.max(z, axis=1)
        zm3 = zm_u[:, None, :]
        kk_u = jnp.min(jnp.where(z == zm3, kio3, K), axis=1)
        flz_u = jnp.sum(jnp.where(kio3 == kk_u[:, None, :], fl3, 0.0), axis=1)
        e_u = ef_ref[pl.ds(d, U), :]
        w_u = jnp.exp(-thc_u) * (e_u - jnp.exp(thc_u))
        sf = sf + jnp.sum(w_u, axis=0, keepdims=True)
        thc_ref[pl.ds(d, U), :] = thc_u
        zmc = jnp.max(zm_u, axis=0, keepdims=True)
        dsel = dio_u == jnp.min(jnp.where(zm_u == zmc, dio_u, U), axis=0,
                                keepdims=True)
        kkc = jnp.sum(jnp.where(dsel, kk_u, 0), axis=0, keepdims=True)
        flc = jnp.sum(jnp.where(dsel, flz_u, 0.0), axis=0, keepdims=True)
        ddc = jnp.sum(jnp.where(dsel, dio_u, 0), axis=0, keepdims=True)
        upd = zmc > bv
        bv = jnp.where(upd, zmc, bv)
        bi = jnp.where(upd, (d + ddc) * K + kkc, bi)
        bfl = jnp.where(upd, flc, bfl)
        return bv, bi, bfl, sf

    ninf = jnp.full((1, B), -jnp.inf, jnp.float32)
    zf = jnp.zeros((1, B), jnp.float32)
    zi = jnp.zeros((1, B), jnp.int32)
    bv_t, bi_t, bfl_t, sf_t = lax.fori_loop(0, DT // U, chunk,
                                            (ninf, zi, zf, zf))

    upd = bv_t > bv_ref[0:1, :]
    bv_ref[0:1, :] = jnp.where(upd, bv_t, bv_ref[0:1, :])
    bi_ref[0:1, :] = jnp.where(upd, bi_t, bi_ref[0:1, :])
    bfl_ref[0:1, :] = jnp.where(upd, bfl_t, bfl_ref[0:1, :])
    sf_ref[0:1, :] = sf_ref[0:1, :] + sf_t

    @pl.when((s == _STEPS - 1) & (t == NT - 1))
    def _fin_last():
        finalize(s)


def kernel(x, theta):
    B, D, K = x.shape
    DT = min(128, D)
    NT = D // DT

    key = jax.random.key(42)
    ksl, kul = [], []
    for _ in range(_STEPS):
        key, ks, ku = jax.random.split(key, 3)
        ksl.append(jax.random.key_data(ks))
        kul.append(jax.random.key_data(ku))
    ksd = jnp.stack(ksl).astype(jnp.uint32)
    kud = jnp.stack(kul).astype(jnp.uint32)

    cur0 = pl.pallas_call(
        functools.partial(_encode_body, K=K),
        grid=(NT,),
        in_specs=[pl.BlockSpec((B, DT, K), lambda t: (0, t, 0))],
        out_specs=pl.BlockSpec((DT, B), lambda t: (t, 0)),
        out_shape=jax.ShapeDtypeStruct((D, B), jnp.int32),
    )(x)

    cur = pl.pallas_call(
        functools.partial(_main_body, B=B, D=D, K=K, DT=DT, NT=NT),
        grid=(_STEPS, NT),
        in_specs=[
            pl.BlockSpec(memory_space=pltpu.SMEM),
            pl.BlockSpec(memory_space=pltpu.SMEM),
            pl.BlockSpec((D, K), lambda s, t: (0, 0)),
            pl.BlockSpec((D, B), lambda s, t: (0, 0)),
        ],
        out_specs=pl.BlockSpec((D, B), lambda s, t: (0, 0)),
        out_shape=jax.ShapeDtypeStruct((D, B), jnp.int32),
        scratch_shapes=[
            pltpu.VMEM((D, B), jnp.float32),
            pltpu.VMEM((D, B), jnp.float32),
            pltpu.VMEM((8, B), jnp.float32),
            pltpu.VMEM((8, B), jnp.int32),
            pltpu.VMEM((8, B), jnp.float32),
            pltpu.VMEM((8, B), jnp.float32),
            pltpu.VMEM((D, K), jnp.float32),
        ],
    )(ksd, kud, theta, cur0)

    x_out = pl.pallas_call(
        functools.partial(_decode_body, K=K),
        grid=(NT,),
        in_specs=[pl.BlockSpec((DT, B), lambda t: (t, 0))],
        out_specs=pl.BlockSpec((B, DT, K), lambda t: (0, t, 0)),
        out_shape=jax.ShapeDtypeStruct((B, D, K), jnp.float32),
    )(cur)

    return x_out
